# baseline (speedup 1.0000x reference)
import functools

import jax
import jax.numpy as jnp
from jax import lax
from jax.experimental import pallas as pl
from jax.experimental.pallas import tpu as pltpu
from jax.experimental.pallas import tpu_sc as plsc

VOCAB = 1000000
EMBED = 32
B = 16384
MAX_SUBWORDS = 10

NC = 2
NS = 16
NW = NC * NS
BPW = B // NW
C = 128
NCHUNK = BPW // C
G = MAX_SUBWORDS


def _body(table_hbm, ids_hbm, lens_hbm, out_hbm, idx_v, rows_v, lens_v,
          out_v, sem):
    wid = lax.axis_index("s") * NC + lax.axis_index("c")
    for chunk in range(NCHUNK):
        wbase = wid * BPW + chunk * C
        pltpu.sync_copy(ids_hbm.at[wid * NCHUNK + chunk], idx_v)
        pltpu.sync_copy(lens_hbm.at[pl.ds(wbase, C)], lens_v)
        copies = [
            pltpu.async_copy(table_hbm.at[idx_v.at[g]],
                             rows_v.at[pl.ds(g * C, C)], sem)
            for g in range(G)
        ]
        for cp in copies:
            cp.wait()

        for q in range(C // 16):
            lens16 = lens_v[pl.ds(q * 16, 16)].astype(jnp.float32)
            linv16 = 1.0 / lens16

            def word_body(t, _, lens16=lens16, linv16=linv16, q=q):
                tf = jnp.full((16,), t, jnp.int32)
                lb = lens16.at[tf].get(mode="promise_in_bounds")
                li = linv16.at[tf].get(mode="promise_in_bounds")
                b_local = q * 16 + t
                base = b_local * MAX_SUBWORDS
                acc0 = jnp.zeros((16,), jnp.float32)
                acc1 = jnp.zeros((16,), jnp.float32)
                for j in range(MAX_SUBWORDS):
                    m = jnp.where(lb > j, 1.0, 0.0)
                    acc0 = acc0 + rows_v[base + j, pl.ds(0, 16)] * m
                    acc1 = acc1 + rows_v[base + j, pl.ds(16, 16)] * m
                out_v[b_local, pl.ds(0, 16)] = acc0 * li
                out_v[b_local, pl.ds(16, 16)] = acc1 * li
                return 0

            lax.fori_loop(0, 16, word_body, 0)
        pltpu.sync_copy(out_v, out_hbm.at[pl.ds(wbase, C)])


@functools.partial(jax.jit, static_argnames=())
def kernel(subword_ids, lengths, table):
    ids3d = subword_ids.reshape(NW * NCHUNK, G, C)
    mesh = plsc.VectorSubcoreMesh(core_axis_name="c", subcore_axis_name="s")
    fn = pl.kernel(
        _body,
        mesh=mesh,
        out_type=jax.ShapeDtypeStruct((B, EMBED), jnp.float32),
        scratch_types=[
            pltpu.VMEM((G, C), jnp.int32),
            pltpu.VMEM((C * G, EMBED), jnp.float32),
            pltpu.VMEM((C,), jnp.int32),
            pltpu.VMEM((C, EMBED), jnp.float32),
            pltpu.SemaphoreType.DMA,
        ],
        compiler_params=pltpu.CompilerParams(use_tc_tiling_on_sc=False),
    )
    return fn(table, ids3d, lengths)
